# baseline (speedup 1.0000x reference)
import functools

import jax
import jax.numpy as jnp
from jax import lax
from jax.experimental import pallas as pl
from jax.experimental.pallas import tpu as pltpu
from jax.experimental.pallas import tpu_sc as plsc

EMBD = 128
_PREC = lax.Precision.DEFAULT


def _leaky(x):
    return jnp.where(x >= 0, x, 0.01 * x)


def _layernorm(x, g, b, eps=1e-5):
    mu = jnp.mean(x, axis=-1, keepdims=True)
    xc = x - mu
    var = jnp.mean(xc * xc, axis=-1, keepdims=True)
    return xc / jnp.sqrt(var + eps) * g + b


def _dot(x, w):
    return jnp.dot(x, w, precision=_PREC, preferred_element_type=jnp.float32)


def _pack2(lo_f32, hi_f32):
    lo = lax.bitcast_convert_type(lo_f32.astype(jnp.bfloat16),
                                  jnp.uint16).astype(jnp.uint32)
    hi = lax.bitcast_convert_type(hi_f32.astype(jnp.bfloat16),
                                  jnp.uint16).astype(jnp.uint32)
    return lax.bitcast_convert_type(lo | (hi << 16), jnp.float32)


def _unpack_lo(x):
    u = lax.bitcast_convert_type(x, jnp.uint32)
    return lax.bitcast_convert_type(u << 16, jnp.float32)


def _unpack_hi(x):
    u = lax.bitcast_convert_type(x, jnp.uint32)
    return lax.bitcast_convert_type(u & jnp.uint32(0xFFFF0000), jnp.float32)


def _pre_body(bx_ref, it_ref, wla_ref, bla_ref, wra_ref, wlb_ref, blb_ref,
              aa_ref, p2_ref):
    aa_ref[...] = _dot(bx_ref[...], wla_ref[...]) + bla_ref[...]
    ba = _dot(it_ref[...], wra_ref[...])
    ab = _dot(it_ref[...], wlb_ref[...]) + blb_ref[...]
    p2_ref[...] = _pack2(ba, ab)


def _tc_pre_tables(boxes, items, pa, pb):
    n = boxes.shape[0]
    blk = 2000
    grid = n // blk
    full = pl.BlockSpec((EMBD, EMBD), lambda i: (0, 0))
    row = pl.BlockSpec((blk, EMBD), lambda i: (i, 0))
    vec = pl.BlockSpec((1, EMBD), lambda i: (0, 0))
    out_sd = jax.ShapeDtypeStruct((n, EMBD), jnp.float32)
    return pl.pallas_call(
        _pre_body,
        grid=(grid,),
        in_specs=[row, row, full, vec, full, full, vec],
        out_specs=[row, row],
        out_shape=[out_sd, out_sd],
    )(boxes, items, pa['Wl'], pa['bl'].reshape(1, EMBD), pa['Wr'],
      pb['Wl'], pb['bl'].reshape(1, EMBD))


def _edge_body(ef_ref, c1_ref, c2_ref, we_ref, wf_ref, bf_ref, g_ref, b_ref,
               o_ref, *, n_real, unpack1, unpack2):
    m = _dot(ef_ref[...], we_ref[...])
    m = m + unpack1(c1_ref[...]) + unpack2(c2_ref[...])
    m = _layernorm(m, g_ref[...], b_ref[...])
    m = _leaky(m)
    m = _dot(m, wf_ref[...]) + bf_ref[...]
    o_ref[...] = jnp.where(pl.program_id(0) < n_real, m, 0.0)


def _ident(x):
    return x


def _tc_edge_mlp(efeat, c1, c2, we, wf, bf, g, b, unpack1=_ident,
                 unpack2=_ident):
    e = efeat.shape[0]
    e_pad = c1.shape[0]
    blk = 2560
    grid = e_pad // blk
    n_real = e // blk
    row = pl.BlockSpec((blk, EMBD), lambda i: (i, 0))
    ef_row = pl.BlockSpec((blk, EMBD), lambda i: (jnp.minimum(i, n_real - 1), 0))
    full = pl.BlockSpec((EMBD, EMBD), lambda i: (0, 0))
    vec = pl.BlockSpec((1, EMBD), lambda i: (0, 0))
    return pl.pallas_call(
        functools.partial(_edge_body, n_real=n_real, unpack1=unpack1,
                          unpack2=unpack2),
        grid=(grid,),
        in_specs=[ef_row, row, row, full, full, vec, vec, vec],
        out_specs=row,
        out_shape=jax.ShapeDtypeStruct((e_pad, EMBD), jnp.float32),
    )(efeat, c1, c2, we, wf, bf.reshape(1, EMBD), g.reshape(1, EMBD),
      b.reshape(1, EMBD))


def _node_body(p_ref, r_ref, g_ref, b_ref, w1a_ref, w1b_ref, bo1_ref,
               wo2_ref, bo2_ref, o_ref):
    s = p_ref[0] + p_ref[1]
    s = _layernorm(s, g_ref[...], b_ref[...])
    h = _dot(s, w1a_ref[...]) + _dot(r_ref[...], w1b_ref[...]) + bo1_ref[...]
    h = _leaky(h)
    o_ref[...] = _leaky(_dot(h, wo2_ref[...]) + bo2_ref[...])


def _node_fused_body(p_ref, r_ref, g_ref, b_ref, w1a_ref, w1b_ref, bo1_ref,
                     wo2_ref, bo2_ref, wrn_ref, o_ref, t_ref):
    s = p_ref[0] + p_ref[1]
    s = _layernorm(s, g_ref[...], b_ref[...])
    h = _dot(s, w1a_ref[...]) + _dot(r_ref[...], w1b_ref[...]) + bo1_ref[...]
    h = _leaky(h)
    h = _leaky(_dot(h, wo2_ref[...]) + bo2_ref[...])
    o_ref[...] = h
    t_ref[...] = _dot(h, wrn_ref[...])


def _tc_node_mlp(partials, right, p, wr_next=None):
    n = right.shape[0]
    blk = 2000
    grid = n // blk
    pspec = pl.BlockSpec((2, blk, EMBD), lambda i: (0, i, 0))
    row = pl.BlockSpec((blk, EMBD), lambda i: (i, 0))
    full = pl.BlockSpec((EMBD, EMBD), lambda i: (0, 0))
    vec = pl.BlockSpec((1, EMBD), lambda i: (0, 0))
    out_sd = jax.ShapeDtypeStruct((n, EMBD), jnp.float32)
    args = (partials, right, p['ln2_g'].reshape(1, EMBD),
            p['ln2_b'].reshape(1, EMBD), p['Wo1'][:EMBD], p['Wo1'][EMBD:],
            p['bo1'].reshape(1, EMBD), p['Wo2'], p['bo2'].reshape(1, EMBD))
    in_specs = [pspec, row, vec, vec, full, full, vec, full, vec]
    if wr_next is None:
        return pl.pallas_call(
            _node_body, grid=(grid,), in_specs=in_specs, out_specs=row,
            out_shape=out_sd)(*args)
    return pl.pallas_call(
        _node_fused_body, grid=(grid,), in_specs=in_specs + [full],
        out_specs=[row, row], out_shape=[out_sd, out_sd],
    )(*args, wr_next)


@functools.cache
def _sc_mesh():
    return plsc.VectorSubcoreMesh(core_axis_name="core",
                                  subcore_axis_name="subcore")


_GATHER_W = 128


def _sc_gather_half(table, idx2):
    n = table.shape[0]
    e = idx2.shape[0] * idx2.shape[1]
    w = _GATHER_W
    chunks = e // (32 * w)
    tstripe = 1000
    n_t = n // tstripe
    seg = 40
    out_sd = jax.ShapeDtypeStruct((e, EMBD), jnp.float32)

    @functools.partial(
        pl.kernel, out_type=out_sd, mesh=_sc_mesh(),
        scratch_types=[pltpu.VMEM_SHARED((n, EMBD), jnp.float32),
                       pltpu.VMEM((seg, w), jnp.int32),
                       pltpu.VMEM((w, EMBD), jnp.float32),
                       pltpu.VMEM((w, EMBD), jnp.float32),
                       pltpu.SemaphoreType.DMA,
                       pltpu.SemaphoreType.DMA])
    def k(t_hbm, i_hbm, cc_hbm, tsh, idxv, buf0, buf1, sem0, sem1):
        cid = lax.axis_index("core")
        sid = lax.axis_index("subcore")
        wid = cid * 16 + sid

        @pl.when(sid < n_t)
        def _():
            pltpu.sync_copy(t_hbm.at[pl.ds(sid * tstripe, tstripe)],
                            tsh.at[pl.ds(sid * tstripe, tstripe)])

        plsc.subcore_barrier()

        base = wid * chunks * w

        @pl.loop(0, chunks // seg)
        def _(t):
            pltpu.sync_copy(
                i_hbm.at[pl.ds(wid * chunks + t * seg, seg)], idxv)

            @pl.loop(0, seg, step=2)
            def _(j):
                for off, buf, sem in ((0, buf0, sem0), (1, buf1, sem1)):
                    jj = t * seg + j + off

                    @pl.when(jj >= 2)
                    def _():
                        pltpu.make_async_copy(
                            buf, cc_hbm.at[pl.ds(base + (jj - 2) * w, w)],
                            sem).wait()

                    pltpu.sync_copy(tsh.at[idxv.at[j + off]], buf)
                    pltpu.async_copy(
                        buf, cc_hbm.at[pl.ds(base + jj * w, w)], sem)

        pltpu.make_async_copy(
            buf0, cc_hbm.at[pl.ds(base + (chunks - 2) * w, w)], sem0).wait()
        pltpu.make_async_copy(
            buf1, cc_hbm.at[pl.ds(base + (chunks - 1) * w, w)], sem1).wait()

    return k(table, idx2)


def _sc_scatter_add(m2, idx2, n_rows):
    e = idx2.shape[0] * idx2.shape[1]
    w = _GATHER_W
    chunks = e // (32 * w)
    seg = 40
    stripe = 1000
    n_z = n_rows // stripe
    zeros = jnp.zeros((n_rows, EMBD), jnp.float32)
    out_sd = jax.ShapeDtypeStruct((2, n_rows, EMBD), jnp.float32)

    @functools.partial(
        pl.kernel, out_type=out_sd, mesh=_sc_mesh(),
        scratch_types=[pltpu.VMEM_SHARED((n_rows, EMBD), jnp.float32),
                       pltpu.VMEM((seg, w), jnp.int32),
                       pltpu.VMEM((w, EMBD), jnp.float32),
                       pltpu.VMEM((w, EMBD), jnp.float32),
                       pltpu.SemaphoreType.DMA,
                       pltpu.SemaphoreType.DMA])
    def k(m_hbm, i_hbm, z_hbm, o_hbm, acc, idxv, buf0, buf1, sem0, sem1):
        cid = lax.axis_index("core")
        sid = lax.axis_index("subcore")
        wid = cid * 16 + sid

        @pl.when(sid < n_z)
        def _():
            pltpu.sync_copy(z_hbm.at[pl.ds(sid * stripe, stripe)],
                            acc.at[pl.ds(sid * stripe, stripe)])

        plsc.subcore_barrier()

        base = wid * chunks

        pltpu.async_copy(m_hbm.at[pl.ds(base * w, w)], buf0, sem0)
        pltpu.async_copy(m_hbm.at[pl.ds((base + 1) * w, w)], buf1, sem1)

        @pl.loop(0, chunks // seg)
        def _(t):
            pltpu.sync_copy(i_hbm.at[pl.ds(base + t * seg, seg)], idxv)

            @pl.loop(0, seg, step=2)
            def _(j):
                for off, buf, sem in ((0, buf0, sem0), (1, buf1, sem1)):
                    jj = t * seg + j + off
                    pltpu.make_async_copy(
                        m_hbm.at[pl.ds((base + jj) * w, w)], buf, sem).wait()
                    pltpu.sync_copy(buf, acc.at[idxv.at[j + off]], add=True)

                    @pl.when(jj + 2 < chunks)
                    def _():
                        pltpu.async_copy(
                            m_hbm.at[pl.ds((base + jj + 2) * w, w)], buf,
                            sem)

        plsc.subcore_barrier()

        @pl.when(sid < n_z)
        def _():
            pltpu.sync_copy(acc.at[pl.ds(sid * stripe, stripe)],
                            o_hbm.at[cid, pl.ds(sid * stripe, stripe)])

    return k(m2, idx2, zeros)


def kernel(items_feats, edge_indices, edge_features, boxes_feats, params_a,
           params_b):
    e = edge_indices.shape[1]
    n = boxes_feats.shape[0]
    unit = 20480
    e_pad = ((e + unit - 1) // unit) * unit
    idx = edge_indices.astype(jnp.int32)
    idx = jnp.pad(idx, ((0, 0), (0, e_pad - e)))
    src = idx[0].reshape(1, -1)
    dst = idx[1].reshape(1, -1)
    src2 = src.reshape(-1, _GATHER_W)
    dst2 = dst.reshape(-1, _GATHER_W)

    tab_aa, p2 = _tc_pre_tables(boxes_feats, items_feats, params_a, params_b)

    c1a = _sc_gather_half(tab_aa, dst2)
    g2 = _sc_gather_half(p2, src2)
    m2a = _tc_edge_mlp(edge_features, c1a, g2, params_a['We'],
                       params_a['Wf'], params_a['bf'], params_a['ln1_g'],
                       params_a['ln1_b'], unpack2=_unpack_lo)
    part_a = _sc_scatter_add(m2a, dst2, n)
    proc_box, tab_bb = _tc_node_mlp(part_a, boxes_feats, params_a,
                                    wr_next=params_b['Wr'])

    c2b = _sc_gather_half(tab_bb, dst2)
    m2b = _tc_edge_mlp(edge_features, g2, c2b, params_b['We'],
                       params_b['Wf'], params_b['bf'], params_b['ln1_g'],
                       params_b['ln1_b'], unpack1=_unpack_hi)
    part_b = _sc_scatter_add(m2b, src2, items_feats.shape[0])
    proc_item = _tc_node_mlp(part_b, items_feats, params_b)

    return (proc_item, proc_box)


# speedup vs baseline: 1.0329x; 1.0329x over previous
import functools

import jax
import jax.numpy as jnp
from jax import lax
from jax.experimental import pallas as pl
from jax.experimental.pallas import tpu as pltpu
from jax.experimental.pallas import tpu_sc as plsc

EMBD = 128
_PREC = lax.Precision.DEFAULT


def _leaky(x):
    return jnp.where(x >= 0, x, 0.01 * x)


def _layernorm(x, g, b, eps=1e-5):
    mu = jnp.mean(x, axis=-1, keepdims=True)
    xc = x - mu
    var = jnp.mean(xc * xc, axis=-1, keepdims=True)
    return xc / jnp.sqrt(var + eps) * g + b


def _dot(x, w):
    return jnp.dot(x, w, precision=_PREC, preferred_element_type=jnp.float32)


def _pack2(lo_f32, hi_f32):
    lo = lax.bitcast_convert_type(lo_f32.astype(jnp.bfloat16),
                                  jnp.uint16).astype(jnp.uint32)
    hi = lax.bitcast_convert_type(hi_f32.astype(jnp.bfloat16),
                                  jnp.uint16).astype(jnp.uint32)
    return lax.bitcast_convert_type(lo | (hi << 16), jnp.float32)


def _unpack_lo(x):
    u = lax.bitcast_convert_type(x, jnp.uint32)
    return lax.bitcast_convert_type(u << 16, jnp.float32)


def _unpack_hi(x):
    u = lax.bitcast_convert_type(x, jnp.uint32)
    return lax.bitcast_convert_type(u & jnp.uint32(0xFFFF0000), jnp.float32)


def _pre_body(bx_ref, it_ref, wla_ref, bla_ref, wra_ref, wlb_ref, blb_ref,
              aa_ref, p2_ref):
    aa_ref[...] = _dot(bx_ref[...], wla_ref[...]) + bla_ref[...]
    ba = _dot(it_ref[...], wra_ref[...])
    ab = _dot(it_ref[...], wlb_ref[...]) + blb_ref[...]
    p2_ref[...] = _pack2(ba, ab)


def _tc_pre_tables(boxes, items, pa, pb):
    n = boxes.shape[0]
    blk = 2000
    grid = n // blk
    full = pl.BlockSpec((EMBD, EMBD), lambda i: (0, 0))
    row = pl.BlockSpec((blk, EMBD), lambda i: (i, 0))
    vec = pl.BlockSpec((1, EMBD), lambda i: (0, 0))
    out_sd = jax.ShapeDtypeStruct((n, EMBD), jnp.float32)
    return pl.pallas_call(
        _pre_body,
        grid=(grid,),
        in_specs=[row, row, full, vec, full, full, vec],
        out_specs=[row, row],
        out_shape=[out_sd, out_sd],
    )(boxes, items, pa['Wl'], pa['bl'].reshape(1, EMBD), pa['Wr'],
      pb['Wl'], pb['bl'].reshape(1, EMBD))


def _edge_body(ef_ref, c1_ref, c2_ref, we_ref, wf_ref, bf_ref, g_ref, b_ref,
               o_ref, *, n_real, unpack1, unpack2):
    m = _dot(ef_ref[...], we_ref[...])
    m = m + unpack1(c1_ref[...]) + unpack2(c2_ref[...])
    m = _layernorm(m, g_ref[...], b_ref[...])
    m = _leaky(m)
    m = _dot(m, wf_ref[...]) + bf_ref[...]
    o_ref[...] = jnp.where(pl.program_id(0) < n_real, m, 0.0)


def _ident(x):
    return x


def _tc_edge_mlp(efeat, c1, c2, we, wf, bf, g, b, unpack1=_ident,
                 unpack2=_ident, i0=0, nblk=None):
    e = efeat.shape[0]
    e_pad = c1.shape[0]
    blk = 2560
    if nblk is None:
        nblk = e_pad // blk
    n_real = e // blk
    row = pl.BlockSpec((blk, EMBD), lambda i: (i + i0, 0))
    orow = pl.BlockSpec((blk, EMBD), lambda i: (i, 0))
    ef_row = pl.BlockSpec(
        (blk, EMBD), lambda i: (jnp.minimum(i + i0, n_real - 1), 0))
    full = pl.BlockSpec((EMBD, EMBD), lambda i: (0, 0))
    vec = pl.BlockSpec((1, EMBD), lambda i: (0, 0))
    return pl.pallas_call(
        functools.partial(_edge_body, n_real=n_real - i0, unpack1=unpack1,
                          unpack2=unpack2),
        grid=(nblk,),
        in_specs=[ef_row, row, row, full, full, vec, vec, vec],
        out_specs=orow,
        out_shape=jax.ShapeDtypeStruct((nblk * blk, EMBD), jnp.float32),
    )(efeat, c1, c2, we, wf, bf.reshape(1, EMBD), g.reshape(1, EMBD),
      b.reshape(1, EMBD))


def _node_body(p_ref, q_ref, r_ref, g_ref, b_ref, w1a_ref, w1b_ref, bo1_ref,
               wo2_ref, bo2_ref, o_ref):
    s = (p_ref[0] + p_ref[1]) + (q_ref[0] + q_ref[1])
    s = _layernorm(s, g_ref[...], b_ref[...])
    h = _dot(s, w1a_ref[...]) + _dot(r_ref[...], w1b_ref[...]) + bo1_ref[...]
    h = _leaky(h)
    o_ref[...] = _leaky(_dot(h, wo2_ref[...]) + bo2_ref[...])


def _node_fused_body(p_ref, q_ref, r_ref, g_ref, b_ref, w1a_ref, w1b_ref,
                     bo1_ref, wo2_ref, bo2_ref, wrn_ref, o_ref, t_ref):
    s = (p_ref[0] + p_ref[1]) + (q_ref[0] + q_ref[1])
    s = _layernorm(s, g_ref[...], b_ref[...])
    h = _dot(s, w1a_ref[...]) + _dot(r_ref[...], w1b_ref[...]) + bo1_ref[...]
    h = _leaky(h)
    h = _leaky(_dot(h, wo2_ref[...]) + bo2_ref[...])
    o_ref[...] = h
    t_ref[...] = _dot(h, wrn_ref[...])


def _tc_node_mlp(partials, partials2, right, p, wr_next=None):
    n = right.shape[0]
    blk = 2000
    grid = n // blk
    pspec = pl.BlockSpec((2, blk, EMBD), lambda i: (0, i, 0))
    row = pl.BlockSpec((blk, EMBD), lambda i: (i, 0))
    full = pl.BlockSpec((EMBD, EMBD), lambda i: (0, 0))
    vec = pl.BlockSpec((1, EMBD), lambda i: (0, 0))
    out_sd = jax.ShapeDtypeStruct((n, EMBD), jnp.float32)
    args = (partials, partials2, right, p['ln2_g'].reshape(1, EMBD),
            p['ln2_b'].reshape(1, EMBD), p['Wo1'][:EMBD], p['Wo1'][EMBD:],
            p['bo1'].reshape(1, EMBD), p['Wo2'], p['bo2'].reshape(1, EMBD))
    in_specs = [pspec, pspec, row, vec, vec, full, full, vec, full, vec]
    if wr_next is None:
        return pl.pallas_call(
            _node_body, grid=(grid,), in_specs=in_specs, out_specs=row,
            out_shape=out_sd)(*args)
    return pl.pallas_call(
        _node_fused_body, grid=(grid,), in_specs=in_specs + [full],
        out_specs=[row, row], out_shape=[out_sd, out_sd],
    )(*args, wr_next)


@functools.cache
def _sc_mesh():
    return plsc.VectorSubcoreMesh(core_axis_name="core",
                                  subcore_axis_name="subcore")


_GATHER_W = 128


def _sc_gather_half(table, idx2):
    n = table.shape[0]
    e = idx2.shape[0] * idx2.shape[1]
    w = _GATHER_W
    chunks = e // (32 * w)
    tstripe = 1000
    n_t = n // tstripe
    seg = 40
    out_sd = jax.ShapeDtypeStruct((e, EMBD), jnp.float32)

    @functools.partial(
        pl.kernel, out_type=out_sd, mesh=_sc_mesh(),
        scratch_types=[pltpu.VMEM_SHARED((n, EMBD), jnp.float32),
                       pltpu.VMEM((seg, w), jnp.int32),
                       pltpu.VMEM((w, EMBD), jnp.float32),
                       pltpu.VMEM((w, EMBD), jnp.float32),
                       pltpu.SemaphoreType.DMA,
                       pltpu.SemaphoreType.DMA])
    def k(t_hbm, i_hbm, cc_hbm, tsh, idxv, buf0, buf1, sem0, sem1):
        cid = lax.axis_index("core")
        sid = lax.axis_index("subcore")
        wid = cid * 16 + sid

        @pl.when(sid < n_t)
        def _():
            pltpu.sync_copy(t_hbm.at[pl.ds(sid * tstripe, tstripe)],
                            tsh.at[pl.ds(sid * tstripe, tstripe)])

        plsc.subcore_barrier()

        base = wid * chunks * w

        @pl.loop(0, chunks // seg)
        def _(t):
            pltpu.sync_copy(
                i_hbm.at[pl.ds(wid * chunks + t * seg, seg)], idxv)

            @pl.loop(0, seg, step=2)
            def _(j):
                for off, buf, sem in ((0, buf0, sem0), (1, buf1, sem1)):
                    jj = t * seg + j + off

                    @pl.when(jj >= 2)
                    def _():
                        pltpu.make_async_copy(
                            buf, cc_hbm.at[pl.ds(base + (jj - 2) * w, w)],
                            sem).wait()

                    pltpu.sync_copy(tsh.at[idxv.at[j + off]], buf)
                    pltpu.async_copy(
                        buf, cc_hbm.at[pl.ds(base + jj * w, w)], sem)

        pltpu.make_async_copy(
            buf0, cc_hbm.at[pl.ds(base + (chunks - 2) * w, w)], sem0).wait()
        pltpu.make_async_copy(
            buf1, cc_hbm.at[pl.ds(base + (chunks - 1) * w, w)], sem1).wait()

    return k(table, idx2)


def _sc_scatter_add(m2, idx2, n_rows):
    e = idx2.shape[0] * idx2.shape[1]
    w = _GATHER_W
    chunks = e // (32 * w)
    seg = 40
    stripe = 1000
    n_z = n_rows // stripe
    zeros = jnp.zeros((n_rows, EMBD), jnp.float32)
    out_sd = jax.ShapeDtypeStruct((2, n_rows, EMBD), jnp.float32)

    @functools.partial(
        pl.kernel, out_type=out_sd, mesh=_sc_mesh(),
        scratch_types=[pltpu.VMEM_SHARED((n_rows, EMBD), jnp.float32),
                       pltpu.VMEM((seg, w), jnp.int32),
                       pltpu.VMEM((w, EMBD), jnp.float32),
                       pltpu.VMEM((w, EMBD), jnp.float32),
                       pltpu.SemaphoreType.DMA,
                       pltpu.SemaphoreType.DMA])
    def k(m_hbm, i_hbm, z_hbm, o_hbm, acc, idxv, buf0, buf1, sem0, sem1):
        cid = lax.axis_index("core")
        sid = lax.axis_index("subcore")
        wid = cid * 16 + sid

        @pl.when(sid < n_z)
        def _():
            pltpu.sync_copy(z_hbm.at[pl.ds(sid * stripe, stripe)],
                            acc.at[pl.ds(sid * stripe, stripe)])

        plsc.subcore_barrier()

        base = wid * chunks

        pltpu.async_copy(m_hbm.at[pl.ds(base * w, w)], buf0, sem0)
        pltpu.async_copy(m_hbm.at[pl.ds((base + 1) * w, w)], buf1, sem1)

        @pl.loop(0, chunks // seg)
        def _(t):
            pltpu.sync_copy(i_hbm.at[pl.ds(base + t * seg, seg)], idxv)

            @pl.loop(0, seg, step=2)
            def _(j):
                for off, buf, sem in ((0, buf0, sem0), (1, buf1, sem1)):
                    jj = t * seg + j + off
                    pltpu.make_async_copy(
                        m_hbm.at[pl.ds((base + jj) * w, w)], buf, sem).wait()
                    pltpu.sync_copy(buf, acc.at[idxv.at[j + off]], add=True)

                    @pl.when(jj + 2 < chunks)
                    def _():
                        pltpu.async_copy(
                            m_hbm.at[pl.ds((base + jj + 2) * w, w)], buf,
                            sem)

        plsc.subcore_barrier()

        @pl.when(sid < n_z)
        def _():
            pltpu.sync_copy(acc.at[pl.ds(sid * stripe, stripe)],
                            o_hbm.at[cid, pl.ds(sid * stripe, stripe)])

    return k(m2, idx2, zeros)


def kernel(items_feats, edge_indices, edge_features, boxes_feats, params_a,
           params_b):
    e = edge_indices.shape[1]
    n = boxes_feats.shape[0]
    unit = 20480
    e_pad = ((e + unit - 1) // unit) * unit
    idx = edge_indices.astype(jnp.int32)
    idx = jnp.pad(idx, ((0, 0), (0, e_pad - e)))
    src = idx[0].reshape(1, -1)
    dst = idx[1].reshape(1, -1)
    src2 = src.reshape(-1, _GATHER_W)
    dst2 = dst.reshape(-1, _GATHER_W)

    tab_aa, p2 = _tc_pre_tables(boxes_feats, items_feats, params_a, params_b)

    c1a = _sc_gather_half(tab_aa, dst2)
    g2 = _sc_gather_half(p2, src2)
    halfb = dst2.shape[0] // 2
    nb2 = e_pad // 2560 // 2
    m2a1 = _tc_edge_mlp(edge_features, c1a, g2, params_a['We'],
                        params_a['Wf'], params_a['bf'], params_a['ln1_g'],
                        params_a['ln1_b'], unpack2=_unpack_lo, i0=0, nblk=nb2)
    part_a1 = _sc_scatter_add(m2a1, dst2[:halfb], n)
    m2a2 = _tc_edge_mlp(edge_features, c1a, g2, params_a['We'],
                        params_a['Wf'], params_a['bf'], params_a['ln1_g'],
                        params_a['ln1_b'], unpack2=_unpack_lo, i0=nb2,
                        nblk=nb2)
    part_a2 = _sc_scatter_add(m2a2, dst2[halfb:], n)
    proc_box, tab_bb = _tc_node_mlp(part_a1, part_a2, boxes_feats, params_a,
                                    wr_next=params_b['Wr'])

    c2b = _sc_gather_half(tab_bb, dst2)
    m2b1 = _tc_edge_mlp(edge_features, g2, c2b, params_b['We'],
                        params_b['Wf'], params_b['bf'], params_b['ln1_g'],
                        params_b['ln1_b'], unpack1=_unpack_hi, i0=0, nblk=nb2)
    part_b1 = _sc_scatter_add(m2b1, src2[:halfb], items_feats.shape[0])
    m2b2 = _tc_edge_mlp(edge_features, g2, c2b, params_b['We'],
                        params_b['Wf'], params_b['bf'], params_b['ln1_g'],
                        params_b['ln1_b'], unpack1=_unpack_hi, i0=nb2,
                        nblk=nb2)
    part_b2 = _sc_scatter_add(m2b2, src2[halfb:], items_feats.shape[0])
    proc_item = _tc_node_mlp(part_b1, part_b2, items_feats, params_b)

    return (proc_item, proc_box)
